# initial kernel scaffold (unmeasured)
import jax
import jax.numpy as jnp
from jax import lax
from jax.experimental import pallas as pl
from jax.experimental.pallas import tpu as pltpu


def kernel(Q, K, V):
    b, s, h, d = Q.shape
    scale = d ** -0.5

    def body(q_ref, k_ref, v_ref, o_ref, k_other, v_other, send_sems, recv_sems):
        my_x = lax.axis_index("x")
        my_y = lax.axis_index("y")
        my_z = lax.axis_index("z")
        partner = (my_x, 1 - my_y, my_z)

        barrier_sem = pltpu.get_barrier_semaphore()
        pl.semaphore_signal(
            barrier_sem, inc=1,
            device_id=partner, device_id_type=pl.DeviceIdType.MESH,
        )
        pl.semaphore_wait(barrier_sem, 1)

        rdma_k = pltpu.make_async_remote_copy(
            src_ref=k_ref, dst_ref=k_other,
            send_sem=send_sems.at[0], recv_sem=recv_sems.at[0],
            device_id=partner, device_id_type=pl.DeviceIdType.MESH,
        )
        rdma_v = pltpu.make_async_remote_copy(
            src_ref=v_ref, dst_ref=v_other,
            send_sem=send_sems.at[1], recv_sem=recv_sems.at[1],
            device_id=partner, device_id_type=pl.DeviceIdType.MESH,
        )
        rdma_k.start()
        rdma_v.start()
        rdma_k.wait()
        rdma_v.wait()

        for bi in range(b):
            for hi in range(h):
                q = q_ref[bi, :, hi, :] * scale
                k1 = k_ref[bi, :, hi, :]
                k2 = k_other[bi, :, hi, :]
                s1 = lax.dot_general(
                    q, k1, (((1,), (1,)), ((), ())),
                    preferred_element_type=jnp.float32,
                )
                s2 = lax.dot_general(
                    q, k2, (((1,), (1,)), ((), ())),
                    preferred_element_type=jnp.float32,
                )
                m = jnp.maximum(
                    jnp.max(s1, axis=1, keepdims=True),
                    jnp.max(s2, axis=1, keepdims=True),
                )
                p1 = jnp.exp(s1 - m)
                p2 = jnp.exp(s2 - m)
                denom = (
                    jnp.sum(p1, axis=1, keepdims=True)
                    + jnp.sum(p2, axis=1, keepdims=True)
                )
                o1 = lax.dot_general(
                    p1, v_ref[bi, :, hi, :], (((1,), (0,)), ((), ())),
                    preferred_element_type=jnp.float32,
                )
                o2 = lax.dot_general(
                    p2, v_other[bi, :, hi, :], (((1,), (0,)), ((), ())),
                    preferred_element_type=jnp.float32,
                )
                o_ref[bi, :, hi, :] = (o1 + o2) / denom

    return pl.pallas_call(
        body,
        out_shape=jax.ShapeDtypeStruct((b, s, h, d), jnp.float32),
        in_specs=[pl.BlockSpec(memory_space=pltpu.VMEM)] * 3,
        out_specs=pl.BlockSpec(memory_space=pltpu.VMEM),
        scratch_shapes=[
            pltpu.VMEM((b, s, h, d), jnp.float32),
            pltpu.VMEM((b, s, h, d), jnp.float32),
            pltpu.SemaphoreType.DMA((2,)),
            pltpu.SemaphoreType.DMA((2,)),
        ],
        compiler_params=pltpu.CompilerParams(collective_id=0),
    )(Q, K, V)


# baseline (device time: 135664 ns/iter reference)
import jax
import jax.numpy as jnp
from jax import lax
from jax.experimental import pallas as pl
from jax.experimental.pallas import tpu as pltpu


def kernel(Q, K, V):
    b, s, h, d = Q.shape
    scale = d ** -0.5

    def body(q_ref, k_ref, v_ref, o_ref, k_other, v_other, send_sems, recv_sems):
        my_x = lax.axis_index("x")
        my_y = lax.axis_index("y")
        my_z = lax.axis_index("z")
        partner = (my_x, 1 - my_y, my_z)

        barrier_sem = pltpu.get_barrier_semaphore()
        pl.semaphore_signal(
            barrier_sem, inc=1,
            device_id=partner, device_id_type=pl.DeviceIdType.MESH,
        )
        pl.semaphore_wait(barrier_sem, 1)

        rdma_k = pltpu.make_async_remote_copy(
            src_ref=k_ref, dst_ref=k_other,
            send_sem=send_sems.at[0], recv_sem=recv_sems.at[0],
            device_id=partner, device_id_type=pl.DeviceIdType.MESH,
        )
        rdma_v = pltpu.make_async_remote_copy(
            src_ref=v_ref, dst_ref=v_other,
            send_sem=send_sems.at[1], recv_sem=recv_sems.at[1],
            device_id=partner, device_id_type=pl.DeviceIdType.MESH,
        )
        rdma_k.start()
        rdma_v.start()
        rdma_k.wait()
        rdma_v.wait()

        for bi in range(b):
            for hi in range(h):
                q = q_ref[bi, :, hi, :] * scale
                k1 = k_ref[bi, :, hi, :]
                k2 = k_other[bi, :, hi, :]
                s1 = lax.dot_general(
                    q, k1, (((1,), (1,)), ((), ())),
                    preferred_element_type=jnp.float32,
                )
                s2 = lax.dot_general(
                    q, k2, (((1,), (1,)), ((), ())),
                    preferred_element_type=jnp.float32,
                )
                m = jnp.maximum(
                    jnp.max(s1, axis=1, keepdims=True),
                    jnp.max(s2, axis=1, keepdims=True),
                )
                p1 = jnp.exp(s1 - m)
                p2 = jnp.exp(s2 - m)
                denom = (
                    jnp.sum(p1, axis=1, keepdims=True)
                    + jnp.sum(p2, axis=1, keepdims=True)
                )
                o1 = lax.dot_general(
                    p1, v_ref[bi, :, hi, :], (((1,), (0,)), ((), ())),
                    preferred_element_type=jnp.float32,
                )
                o2 = lax.dot_general(
                    p2, v_other[bi, :, hi, :], (((1,), (0,)), ((), ())),
                    preferred_element_type=jnp.float32,
                )
                o_ref[bi, :, hi, :] = (o1 + o2) / denom

    return pl.pallas_call(
        body,
        out_shape=jax.ShapeDtypeStruct((b, s, h, d), jnp.float32),
        in_specs=[pl.BlockSpec(memory_space=pltpu.VMEM)] * 3,
        out_specs=pl.BlockSpec(memory_space=pltpu.VMEM),
        scratch_shapes=[
            pltpu.VMEM((b, s, h, d), jnp.float32),
            pltpu.VMEM((b, s, h, d), jnp.float32),
            pltpu.SemaphoreType.DMA((2,)),
            pltpu.SemaphoreType.DMA((2,)),
        ],
        compiler_params=pltpu.CompilerParams(
            collective_id=0, vmem_limit_bytes=100 * 1024 * 1024
        ),
    )(Q, K, V)


# device time: 124283 ns/iter; 1.0916x vs baseline; 1.0916x over previous
import jax
import jax.numpy as jnp
from jax import lax
from jax.experimental import pallas as pl
from jax.experimental.pallas import tpu as pltpu


def kernel(Q, K, V):
    b, s, h, d = Q.shape
    scale = d ** -0.5

    Qt = jnp.transpose(Q, (0, 2, 1, 3))
    Kt = jnp.transpose(K, (0, 2, 1, 3))
    Vt = jnp.transpose(V, (0, 2, 1, 3))

    def body(q_ref, k_ref, v_ref, o_ref, k_other, v_other, send_sems, recv_sems):
        my_x = lax.axis_index("x")
        my_y = lax.axis_index("y")
        my_z = lax.axis_index("z")
        partner = (my_x, 1 - my_y, my_z)

        barrier_sem = pltpu.get_barrier_semaphore()
        pl.semaphore_signal(
            barrier_sem, inc=1,
            device_id=partner, device_id_type=pl.DeviceIdType.MESH,
        )
        pl.semaphore_wait(barrier_sem, 1)

        rdma_k = pltpu.make_async_remote_copy(
            src_ref=k_ref, dst_ref=k_other,
            send_sem=send_sems.at[0], recv_sem=recv_sems.at[0],
            device_id=partner, device_id_type=pl.DeviceIdType.MESH,
        )
        rdma_v = pltpu.make_async_remote_copy(
            src_ref=v_ref, dst_ref=v_other,
            send_sem=send_sems.at[1], recv_sem=recv_sems.at[1],
            device_id=partner, device_id_type=pl.DeviceIdType.MESH,
        )
        rdma_k.start()
        rdma_v.start()
        rdma_k.wait()
        rdma_v.wait()

        for bi in range(b):
            for hi in range(h):
                q = q_ref[bi, hi] * scale
                k1 = k_ref[bi, hi]
                k2 = k_other[bi, hi]
                s1 = lax.dot_general(
                    q, k1, (((1,), (1,)), ((), ())),
                    preferred_element_type=jnp.float32,
                )
                s2 = lax.dot_general(
                    q, k2, (((1,), (1,)), ((), ())),
                    preferred_element_type=jnp.float32,
                )
                m = jnp.maximum(
                    jnp.max(s1, axis=1, keepdims=True),
                    jnp.max(s2, axis=1, keepdims=True),
                )
                p1 = jnp.exp(s1 - m)
                p2 = jnp.exp(s2 - m)
                denom = (
                    jnp.sum(p1, axis=1, keepdims=True)
                    + jnp.sum(p2, axis=1, keepdims=True)
                )
                o1 = lax.dot_general(
                    p1, v_ref[bi, hi], (((1,), (0,)), ((), ())),
                    preferred_element_type=jnp.float32,
                )
                o2 = lax.dot_general(
                    p2, v_other[bi, hi], (((1,), (0,)), ((), ())),
                    preferred_element_type=jnp.float32,
                )
                o_ref[bi, hi] = (o1 + o2) / denom

    out = pl.pallas_call(
        body,
        out_shape=jax.ShapeDtypeStruct((b, h, s, d), jnp.float32),
        in_specs=[pl.BlockSpec(memory_space=pltpu.VMEM)] * 3,
        out_specs=pl.BlockSpec(memory_space=pltpu.VMEM),
        scratch_shapes=[
            pltpu.VMEM((b, h, s, d), jnp.float32),
            pltpu.VMEM((b, h, s, d), jnp.float32),
            pltpu.SemaphoreType.DMA((2,)),
            pltpu.SemaphoreType.DMA((2,)),
        ],
        compiler_params=pltpu.CompilerParams(
            collective_id=0, vmem_limit_bytes=100 * 1024 * 1024
        ),
    )(Qt, Kt, Vt)
    return jnp.transpose(out, (0, 2, 1, 3))


# device time: 99257 ns/iter; 1.3668x vs baseline; 1.2521x over previous
import jax
import jax.numpy as jnp
from jax import lax
from jax.experimental import pallas as pl
from jax.experimental.pallas import tpu as pltpu

C = 4


def kernel(Q, K, V):
    b, s, h, d = Q.shape
    scale = d ** -0.5
    half = s // 2
    ck = half // C
    nbh = b * h

    Qt = jnp.transpose(Q, (0, 2, 1, 3))
    Kt = jnp.transpose(K, (0, 2, 1, 3))
    Vt = jnp.transpose(V, (0, 2, 1, 3))

    def body(q_ref, k_ref, v_ref, o_ref, d_buf, r_buf, l_ref,
             y_send, y_recv, x_send, x_recv):
        my_x = lax.axis_index("x")
        my_y = lax.axis_index("y")
        my_z = lax.axis_index("z")
        partner = (my_x, 1 - my_y, my_z)
        xnbr = (1 - my_x, my_y, my_z)

        barrier_sem = pltpu.get_barrier_semaphore()
        for nb in (partner, xnbr):
            pl.semaphore_signal(
                barrier_sem, inc=1,
                device_id=nb, device_id_type=pl.DeviceIdType.MESH,
            )
        pl.semaphore_wait(barrier_sem, 2)

        y_rdmas = []
        for c in range(C):
            for t, src in enumerate((k_ref, v_ref)):
                r = pltpu.make_async_remote_copy(
                    src_ref=src.at[:, :, pl.ds(my_x * half + c * ck, ck), :],
                    dst_ref=d_buf.at[t, :, :, pl.ds(c * ck, ck), :],
                    send_sem=y_send.at[2 * c + t],
                    recv_sem=y_recv.at[2 * c + t],
                    device_id=partner, device_id_type=pl.DeviceIdType.MESH,
                )
                r.start()
                y_rdmas.append(r)

        def accum(i, get_k, get_v, init):
            bi = i // h
            hi = i - bi * h
            q = q_ref[bi, hi] * scale
            kk = get_k(bi, hi)
            vv = get_v(bi, hi)
            sc = lax.dot_general(
                q, kk, (((1,), (1,)), ((), ())),
                preferred_element_type=jnp.float32,
            )
            p = jnp.exp(sc)
            lsum = jnp.sum(p, axis=1, keepdims=True)
            o = lax.dot_general(
                p, vv, (((1,), (0,)), ((), ())),
                preferred_element_type=jnp.float32,
            )
            if init:
                l_ref[bi, hi] = lsum
                o_ref[bi, hi] = o
            else:
                l_ref[bi, hi] = l_ref[bi, hi] + lsum
                o_ref[bi, hi] = o_ref[bi, hi] + o
            return 0

        lax.fori_loop(
            0, nbh,
            lambda i, _: accum(
                i, lambda bi, hi: k_ref[bi, hi], lambda bi, hi: v_ref[bi, hi],
                init=True,
            ),
            0,
        )

        x_rdmas = []
        for c in range(C):
            y_rdmas[2 * c + 0].wait_recv()
            y_rdmas[2 * c + 1].wait_recv()
            for t in range(2):
                r = pltpu.make_async_remote_copy(
                    src_ref=d_buf.at[t, :, :, pl.ds(c * ck, ck), :],
                    dst_ref=r_buf.at[t, :, :, pl.ds(c * ck, ck), :],
                    send_sem=x_send.at[2 * c + t],
                    recv_sem=x_recv.at[2 * c + t],
                    device_id=xnbr, device_id_type=pl.DeviceIdType.MESH,
                )
                r.start()
                x_rdmas.append(r)
            lax.fori_loop(
                0, nbh,
                lambda i, _, c=c: accum(
                    i,
                    lambda bi, hi: d_buf[0, bi, hi, pl.ds(c * ck, ck), :],
                    lambda bi, hi: d_buf[1, bi, hi, pl.ds(c * ck, ck), :],
                    init=False,
                ),
                0,
            )

        for c in range(C):
            x_rdmas[2 * c + 0].wait_recv()
            x_rdmas[2 * c + 1].wait_recv()
            lax.fori_loop(
                0, nbh,
                lambda i, _, c=c: accum(
                    i,
                    lambda bi, hi: r_buf[0, bi, hi, pl.ds(c * ck, ck), :],
                    lambda bi, hi: r_buf[1, bi, hi, pl.ds(c * ck, ck), :],
                    init=False,
                ),
                0,
            )

        def fin(i, _):
            bi = i // h
            hi = i - bi * h
            o_ref[bi, hi] = o_ref[bi, hi] / l_ref[bi, hi]
            return 0

        lax.fori_loop(0, nbh, fin, 0)

        for r in y_rdmas + x_rdmas:
            r.wait_send()

    out = pl.pallas_call(
        body,
        out_shape=jax.ShapeDtypeStruct((b, h, s, d), jnp.float32),
        in_specs=[pl.BlockSpec(memory_space=pltpu.VMEM)] * 3,
        out_specs=pl.BlockSpec(memory_space=pltpu.VMEM),
        scratch_shapes=[
            pltpu.VMEM((2, b, h, half, d), jnp.float32),
            pltpu.VMEM((2, b, h, half, d), jnp.float32),
            pltpu.VMEM((b, h, s, 1), jnp.float32),
            pltpu.SemaphoreType.DMA((2 * C,)),
            pltpu.SemaphoreType.DMA((2 * C,)),
            pltpu.SemaphoreType.DMA((2 * C,)),
            pltpu.SemaphoreType.DMA((2 * C,)),
        ],
        compiler_params=pltpu.CompilerParams(
            collective_id=0, vmem_limit_bytes=100 * 1024 * 1024
        ),
    )(Qt, Kt, Vt)
    return jnp.transpose(out, (0, 2, 1, 3))


# device time: 92307 ns/iter; 1.4697x vs baseline; 1.0753x over previous
import jax
import jax.numpy as jnp
from jax import lax
from jax.experimental import pallas as pl
from jax.experimental.pallas import tpu as pltpu

C = 4


def kernel(Q, K, V):
    b, s, h, d = Q.shape
    scale = d ** -0.5
    half = s // 2
    ck = half // C
    nbh = b * h

    Qt = jnp.transpose(Q, (0, 2, 1, 3))
    Kt = jnp.transpose(K, (0, 2, 1, 3))
    Vt = jnp.transpose(V, (0, 2, 1, 3))

    def body(q_ref, k_ref, v_ref, o_ref, d_buf, r_buf, l_ref,
             y_send, y_recv, x_send, x_recv):
        my_x = lax.axis_index("x")
        my_y = lax.axis_index("y")
        my_z = lax.axis_index("z")
        partner = (my_x, 1 - my_y, my_z)
        xnbr = (1 - my_x, my_y, my_z)

        barrier_sem = pltpu.get_barrier_semaphore()
        for nb in (partner, xnbr):
            pl.semaphore_signal(
                barrier_sem, inc=1,
                device_id=nb, device_id_type=pl.DeviceIdType.MESH,
            )
        pl.semaphore_wait(barrier_sem, 2)

        y_rdmas = []
        for c in range(C):
            for t, src in enumerate((k_ref, v_ref)):
                r = pltpu.make_async_remote_copy(
                    src_ref=src.at[:, :, pl.ds(my_x * half + c * ck, ck), :],
                    dst_ref=d_buf.at[t, :, :, pl.ds(c * ck, ck), :],
                    send_sem=y_send.at[2 * c + t],
                    recv_sem=y_recv.at[2 * c + t],
                    device_id=partner, device_id_type=pl.DeviceIdType.MESH,
                )
                r.start()
                y_rdmas.append(r)

        def accum(i, get_k, get_v, init):
            bi = i // h
            hi = i - bi * h
            q = q_ref[bi, hi] * scale
            kk = get_k(bi, hi)
            vv = get_v(bi, hi)
            sc = lax.dot_general(
                q, kk, (((1,), (1,)), ((), ())),
                preferred_element_type=jnp.float32,
            )
            p = jnp.exp(sc)
            lsum = jnp.sum(p, axis=1, keepdims=True)
            o = lax.dot_general(
                p, vv, (((1,), (0,)), ((), ())),
                preferred_element_type=jnp.float32,
            )
            if init:
                l_ref[bi, hi] = lsum
                o_ref[bi, hi] = o
            else:
                l_ref[bi, hi] = l_ref[bi, hi] + lsum
                o_ref[bi, hi] = o_ref[bi, hi] + o
            return 0

        lax.fori_loop(
            0, nbh,
            lambda i, _: accum(
                i, lambda bi, hi: k_ref[bi, hi], lambda bi, hi: v_ref[bi, hi],
                init=True,
            ),
            0,
        )

        x_rdmas = []
        for c in range(C):
            y_rdmas[2 * c + 0].wait_recv()
            y_rdmas[2 * c + 1].wait_recv()
            for t in range(2):
                r = pltpu.make_async_remote_copy(
                    src_ref=d_buf.at[t, :, :, pl.ds(c * ck, ck), :],
                    dst_ref=r_buf.at[t, :, :, pl.ds(c * ck, ck), :],
                    send_sem=x_send.at[2 * c + t],
                    recv_sem=x_recv.at[2 * c + t],
                    device_id=xnbr, device_id_type=pl.DeviceIdType.MESH,
                )
                r.start()
                x_rdmas.append(r)

        lax.fori_loop(
            0, nbh,
            lambda i, _: accum(
                i,
                lambda bi, hi: d_buf[0, bi, hi],
                lambda bi, hi: d_buf[1, bi, hi],
                init=False,
            ),
            0,
        )

        for r in x_rdmas:
            r.wait_recv()
        lax.fori_loop(
            0, nbh,
            lambda i, _: accum(
                i,
                lambda bi, hi: r_buf[0, bi, hi],
                lambda bi, hi: r_buf[1, bi, hi],
                init=False,
            ),
            0,
        )

        def fin(i, _):
            bi = i // h
            hi = i - bi * h
            o_ref[bi, hi] = o_ref[bi, hi] / l_ref[bi, hi]
            return 0

        lax.fori_loop(0, nbh, fin, 0)

        for r in y_rdmas + x_rdmas:
            r.wait_send()

    out = pl.pallas_call(
        body,
        out_shape=jax.ShapeDtypeStruct((b, h, s, d), jnp.float32),
        in_specs=[pl.BlockSpec(memory_space=pltpu.VMEM)] * 3,
        out_specs=pl.BlockSpec(memory_space=pltpu.VMEM),
        scratch_shapes=[
            pltpu.VMEM((2, b, h, half, d), jnp.float32),
            pltpu.VMEM((2, b, h, half, d), jnp.float32),
            pltpu.VMEM((b, h, s, 1), jnp.float32),
            pltpu.SemaphoreType.DMA((2 * C,)),
            pltpu.SemaphoreType.DMA((2 * C,)),
            pltpu.SemaphoreType.DMA((2 * C,)),
            pltpu.SemaphoreType.DMA((2 * C,)),
        ],
        compiler_params=pltpu.CompilerParams(
            collective_id=0, vmem_limit_bytes=100 * 1024 * 1024
        ),
    )(Qt, Kt, Vt)
    return jnp.transpose(out, (0, 2, 1, 3))
